# baseline (speedup 1.0000x reference)
import functools

import jax
import jax.numpy as jnp
from jax import lax
from jax.experimental import pallas as pl
from jax.experimental.pallas import tpu as pltpu
from jax.experimental.pallas import tpu_sc as plsc

_NC = 2
_NS = 16
_NW = _NC * _NS

_NBUF = 4
_CHUNK = 640


def _gather_call(idx_flat, table):
    n, = idx_flat.shape
    _, d = table.shape
    n_per_w = n // _NW
    n_chunks = n_per_w // _CHUNK
    n_rounds = n_chunks // _NBUF
    n_pro = (_NBUF - 1) * _CHUNK

    mesh = plsc.VectorSubcoreMesh(
        core_axis_name="c", subcore_axis_name="s",
        num_cores=_NC, num_subcores=_NS)

    @functools.partial(
        pl.kernel,
        out_type=jax.ShapeDtypeStruct((n, d), jnp.float32),
        mesh=mesh,
        scratch_types=[
            pltpu.VMEM((n_per_w,), jnp.int32),
            [pltpu.VMEM((_CHUNK, d), jnp.float32) for _ in range(_NBUF)],
            [pltpu.SemaphoreType.DMA for _ in range(_NBUF)],
            [pltpu.SemaphoreType.DMA for _ in range(_NBUF)],
            pltpu.SemaphoreType.DMA,
        ],
        compiler_params=pltpu.CompilerParams(use_tc_tiling_on_sc=False),
    )
    def k(idx_hbm, table_hbm, out_hbm, idx_v, rows, gsem, osem, isem):
        wid = lax.axis_index("s") * _NC + lax.axis_index("c")
        base = pl.multiple_of(wid * n_per_w, 8)

        def fire_g(ci, b):
            off = pl.multiple_of(ci * _CHUNK, 8)
            pltpu.async_copy(
                table_hbm.at[idx_v.at[pl.ds(off, _CHUNK)]], rows[b], gsem[b])

        def wait_g(b):
            pltpu.make_async_copy(
                out_hbm.at[pl.ds(base, _CHUNK)], rows[b], gsem[b]).wait()

        def fire_s(ci, b):
            off = pl.multiple_of(ci * _CHUNK, 8)
            pltpu.async_copy(
                rows[b], out_hbm.at[pl.ds(base + off, _CHUNK)], osem[b])

        def wait_s(b):
            pltpu.make_async_copy(
                rows[b], out_hbm.at[pl.ds(base, _CHUNK)], osem[b]).wait()

        pltpu.sync_copy(idx_hbm.at[pl.ds(base, n_pro)],
                        idx_v.at[pl.ds(0, n_pro)])
        for b in range(_NBUF - 1):
            fire_g(b, b)
        rest = pltpu.async_copy(
            idx_hbm.at[pl.ds(base + n_pro, n_per_w - n_pro)],
            idx_v.at[pl.ds(n_pro, n_per_w - n_pro)], isem)

        wait_g(0); fire_s(0, 0)
        rest.wait()
        fire_g(_NBUF - 1, _NBUF - 1)
        for b in range(1, _NBUF):
            wait_g(b)
            fire_s(b, b)
            wait_s(b - 1)
            fire_g(b + _NBUF - 1, b - 1)

        def round_body(r, carry):
            for b in range(_NBUF):
                ci = r * _NBUF + b
                wait_g(b)
                fire_s(ci, b)
                wait_s((b + _NBUF - 1) % _NBUF)
                fire_g(ci + _NBUF - 1, (b + _NBUF - 1) % _NBUF)
            return carry

        lax.fori_loop(1, n_rounds - 1, round_body, 0)

        c0 = (n_rounds - 1) * _NBUF
        wait_g(0); fire_s(c0, 0); wait_s(_NBUF - 1)
        fire_g(c0 + _NBUF - 1, _NBUF - 1)
        for b in range(1, _NBUF):
            wait_g(b)
            fire_s(c0 + b, b)

        for b in range(_NBUF):
            wait_s(b)

    return k(idx_flat, table)


def kernel(token_tag, tok_table, tag_table):
    b, s = token_tag.shape
    _, d = tok_table.shape
    idx = token_tag.reshape(b * s)
    out = _gather_call(idx, tok_table)
    return out.reshape(b, s, d)


# speedup vs baseline: 1.0001x; 1.0001x over previous
import functools

import jax
import jax.numpy as jnp
from jax import lax
from jax.experimental import pallas as pl
from jax.experimental.pallas import tpu as pltpu
from jax.experimental.pallas import tpu_sc as plsc

_NC = 2
_NS = 16
_NW = _NC * _NS

_NBUF = 4
_CHUNK = 320


def _gather_call(idx_flat, table):
    n, = idx_flat.shape
    _, d = table.shape
    n_per_w = n // _NW
    n_chunks = n_per_w // _CHUNK
    n_rounds = n_chunks // _NBUF
    n_pro = (_NBUF - 1) * _CHUNK

    mesh = plsc.VectorSubcoreMesh(
        core_axis_name="c", subcore_axis_name="s",
        num_cores=_NC, num_subcores=_NS)

    @functools.partial(
        pl.kernel,
        out_type=jax.ShapeDtypeStruct((n, d), jnp.float32),
        mesh=mesh,
        scratch_types=[
            pltpu.VMEM((n_per_w,), jnp.int32),
            [pltpu.VMEM((_CHUNK, d), jnp.float32) for _ in range(_NBUF)],
            [pltpu.SemaphoreType.DMA for _ in range(_NBUF)],
            [pltpu.SemaphoreType.DMA for _ in range(_NBUF)],
            pltpu.SemaphoreType.DMA,
        ],
        compiler_params=pltpu.CompilerParams(use_tc_tiling_on_sc=False),
    )
    def k(idx_hbm, table_hbm, out_hbm, idx_v, rows, gsem, osem, isem):
        wid = lax.axis_index("s") * _NC + lax.axis_index("c")
        base = pl.multiple_of(wid * n_per_w, 8)

        def fire_g(ci, b):
            off = pl.multiple_of(ci * _CHUNK, 8)
            pltpu.async_copy(
                table_hbm.at[idx_v.at[pl.ds(off, _CHUNK)]], rows[b], gsem[b])

        def wait_g(b):
            pltpu.make_async_copy(
                out_hbm.at[pl.ds(base, _CHUNK)], rows[b], gsem[b]).wait()

        def fire_s(ci, b):
            off = pl.multiple_of(ci * _CHUNK, 8)
            pltpu.async_copy(
                rows[b], out_hbm.at[pl.ds(base + off, _CHUNK)], osem[b])

        def wait_s(b):
            pltpu.make_async_copy(
                rows[b], out_hbm.at[pl.ds(base, _CHUNK)], osem[b]).wait()

        pltpu.sync_copy(idx_hbm.at[pl.ds(base, n_pro)],
                        idx_v.at[pl.ds(0, n_pro)])
        for b in range(_NBUF - 1):
            fire_g(b, b)
        rest = pltpu.async_copy(
            idx_hbm.at[pl.ds(base + n_pro, n_per_w - n_pro)],
            idx_v.at[pl.ds(n_pro, n_per_w - n_pro)], isem)

        wait_g(0); fire_s(0, 0)
        rest.wait()
        fire_g(_NBUF - 1, _NBUF - 1)
        for b in range(1, _NBUF):
            wait_g(b)
            fire_s(b, b)
            wait_s(b - 1)
            fire_g(b + _NBUF - 1, b - 1)

        def round_body(r, carry):
            for b in range(_NBUF):
                ci = r * _NBUF + b
                wait_g(b)
                fire_s(ci, b)
                wait_s((b + _NBUF - 1) % _NBUF)
                fire_g(ci + _NBUF - 1, (b + _NBUF - 1) % _NBUF)
            return carry

        lax.fori_loop(1, n_rounds - 1, round_body, 0)

        c0 = (n_rounds - 1) * _NBUF
        wait_g(0); fire_s(c0, 0); wait_s(_NBUF - 1)
        fire_g(c0 + _NBUF - 1, _NBUF - 1)
        for b in range(1, _NBUF):
            wait_g(b)
            fire_s(c0 + b, b)

        for b in range(_NBUF):
            wait_s(b)

    return k(idx_flat, table)


def kernel(token_tag, tok_table, tag_table):
    b, s = token_tag.shape
    _, d = tok_table.shape
    idx = token_tag.reshape(b * s)
    out = _gather_call(idx, tok_table)
    return out.reshape(b, s, d)
